# baseline (speedup 1.0000x reference)
import functools

import jax
import jax.numpy as jnp
from jax import lax
from jax.experimental import pallas as pl
from jax.experimental.pallas import tpu as pltpu
from jax.experimental.pallas import tpu_sc as plsc

EMBED_DIM = 64
NUM_CORES = 2
NUM_SUBCORES = 16
NW = NUM_CORES * NUM_SUBCORES
CHUNK = 128
NBUF = 10
LAG = 5

_mesh = plsc.VectorSubcoreMesh(core_axis_name="c", subcore_axis_name="s")


def _make_gather(batch: int, dim: int):
  bpw = batch // NW
  nchunk = bpw // CHUNK
  ngroup = nchunk // NBUF
  assert nchunk % NBUF == 0

  @functools.partial(
      pl.kernel,
      mesh=_mesh,
      compiler_params=pltpu.CompilerParams(use_tc_tiling_on_sc=False),
      out_type=jax.ShapeDtypeStruct((batch, dim), jnp.float32),
      scratch_types=[
          pltpu.VMEM((nchunk, CHUNK), jnp.int32),
          pltpu.VMEM((NBUF, CHUNK, dim), jnp.float32),
      ]
      + [pltpu.SemaphoreType.DMA] * (2 * NBUF),
  )
  def gather_kernel(idx_hbm, table_hbm, out_hbm, idx_v, buf, *sems):
    gsem = sems[:NBUF]
    osem = sems[NBUF:]
    wid = lax.axis_index("s") * NUM_CORES + lax.axis_index("c")
    base = wid * bpw
    pltpu.sync_copy(idx_hbm.at[wid], idx_v)

    def fire(j, b):
      pltpu.async_copy(table_hbm.at[idx_v.at[j]], buf.at[b], gsem[b])

    def drain(j, b):
      pltpu.make_async_copy(
          table_hbm.at[idx_v.at[j]], buf.at[b], gsem[b]
      ).wait()
      pltpu.async_copy(
          buf.at[b], out_hbm.at[pl.ds(base + j * CHUNK, CHUNK)], osem[b]
      )

    def wait_out(j, b):
      pltpu.make_async_copy(
          buf.at[b], out_hbm.at[pl.ds(base + j * CHUNK, CHUNK)], osem[b]
      ).wait()

    def body(g, carry):
      for b in range(NBUF):
        i = g * NBUF + b
        pl.when(g >= 1)(lambda: wait_out(i - NBUF, b))
        fire(i, b)
        b2 = (b + LAG) % NBUF
        if b < LAG:
          pl.when(g >= 1)(lambda: drain(i - LAG, b2))
        else:
          drain(i - LAG, b2)
      return carry

    lax.fori_loop(0, ngroup, body, 0, unroll=False)

    last = ngroup - 1
    for b in range(LAG):
      j = last * NBUF + NBUF - LAG + b
      drain(j, b + NBUF - LAG)
    for b in range(NBUF):
      j = last * NBUF + b
      wait_out(j, b)

  return gather_kernel


def kernel(x, table):
  batch, hist = x.shape
  total = batch * hist
  idx = x.reshape(NW, total // (NW * CHUNK), CHUNK)
  out_neg = _make_gather(total, EMBED_DIM)(idx, jnp.negative(table))
  return jnp.negative(out_neg).reshape(batch, hist, EMBED_DIM)


# speedup vs baseline: 1.2666x; 1.2666x over previous
import functools

import jax
import jax.numpy as jnp
from jax import lax
from jax.experimental import pallas as pl
from jax.experimental.pallas import tpu as pltpu
from jax.experimental.pallas import tpu_sc as plsc

EMBED_DIM = 64
PAD_DIM = 128
LANES = 16
NUM_CORES = 2
NUM_SUBCORES = 16
NW = NUM_CORES * NUM_SUBCORES

_mesh = plsc.VectorSubcoreMesh(core_axis_name="c", subcore_axis_name="s")


def _make_gather(batch: int, hist: int):
  bpw = batch // NW

  @functools.partial(
      pl.kernel,
      mesh=_mesh,
      out_type=jax.ShapeDtypeStruct((batch, hist, EMBED_DIM), jnp.float32),
      scratch_types=[
          pltpu.VMEM((bpw, hist), jnp.int32),
          pltpu.VMEM((hist, PAD_DIM), jnp.float32),
          pltpu.VMEM((hist, EMBED_DIM), jnp.float32),
          pltpu.SemaphoreType.DMA,
      ],
  )
  def gather_kernel(x_hbm, table_hbm, out_hbm, idx_v, buf128, buf64, sem):
    wid = lax.axis_index("s") * NUM_CORES + lax.axis_index("c")
    base = wid * bpw
    pltpu.sync_copy(x_hbm.at[pl.ds(base, bpw)], idx_v)

    def body(j, carry):
      pltpu.async_copy(table_hbm.at[idx_v.at[j]], buf128, sem).wait()
      for r in range(hist):
        for c in range(EMBED_DIM // LANES):
          buf64[r, pl.ds(c * LANES, LANES)] = buf128[r, pl.ds(c * LANES, LANES)]
      pltpu.sync_copy(buf64, out_hbm.at[base + j])
      return carry

    lax.fori_loop(0, bpw, body, 0, unroll=False)

  return gather_kernel


def kernel(x, table):
  batch, hist = x.shape
  table128 = jnp.pad(table, ((0, 0), (0, PAD_DIM - EMBED_DIM)))
  return _make_gather(batch, hist)(x, table128)


# speedup vs baseline: 1.8631x; 1.4709x over previous
import functools

import jax
import jax.numpy as jnp
from jax import lax
from jax.experimental import pallas as pl
from jax.experimental.pallas import tpu as pltpu
from jax.experimental.pallas import tpu_sc as plsc

EMBED_DIM = 64
PAD_DIM = 128
LANES = 16
NUM_CORES = 2
NUM_SUBCORES = 16
NW = NUM_CORES * NUM_SUBCORES
NBUF = 4
LAG = 2

_mesh = plsc.VectorSubcoreMesh(core_axis_name="c", subcore_axis_name="s")


def _make_gather(batch: int, hist: int):
  bpw = batch // NW
  ngroup = bpw // NBUF
  assert bpw % NBUF == 0

  @functools.partial(
      pl.kernel,
      mesh=_mesh,
      out_type=jax.ShapeDtypeStruct((batch, hist, EMBED_DIM), jnp.float32),
      scratch_types=[
          pltpu.VMEM((bpw, hist), jnp.int32),
          pltpu.VMEM((NBUF, hist, PAD_DIM), jnp.float32),
          pltpu.VMEM((NBUF, hist, EMBED_DIM), jnp.float32),
      ]
      + [pltpu.SemaphoreType.DMA] * (2 * NBUF),
  )
  def gather_kernel(x_hbm, table_hbm, out_hbm, idx_v, buf128, buf64, *sems):
    gsem = sems[:NBUF]
    osem = sems[NBUF:]
    wid = lax.axis_index("s") * NUM_CORES + lax.axis_index("c")
    base = wid * bpw
    pltpu.sync_copy(x_hbm.at[pl.ds(base, bpw)], idx_v)

    def fire(j, b):
      pltpu.async_copy(table_hbm.at[idx_v.at[j]], buf128.at[b], gsem[b])

    def wait_gather(j, b):
      pltpu.make_async_copy(
          table_hbm.at[idx_v.at[j]], buf128.at[b], gsem[b]
      ).wait()

    def compact(b):
      for r in range(hist):
        for c in range(EMBED_DIM // LANES):
          sl = pl.ds(c * LANES, LANES)
          buf64[b, r, sl] = buf128[b, r, sl]

    def fire_out(j, b):
      pltpu.async_copy(buf64.at[b], out_hbm.at[base + j], osem[b])

    def wait_out(j, b):
      pltpu.make_async_copy(
          buf64.at[b], out_hbm.at[base + j], osem[b]
      ).wait()

    for b in range(LAG):
      fire(b, b)

    def body(g, carry):
      for b in range(NBUF):
        j = g * NBUF + b
        b2 = (b + LAG) % NBUF
        jf = j + LAG
        if b < NBUF - LAG:
          fire(jf, b2)
        else:
          pl.when(g <= ngroup - 2)(lambda: fire(jf, b2))
        pl.when(g >= 1)(lambda: wait_out(j - NBUF, b))
        wait_gather(j, b)
        compact(b)
        fire_out(j, b)
      return carry

    lax.fori_loop(0, ngroup, body, 0, unroll=False)

    for b in range(NBUF):
      j = (ngroup - 1) * NBUF + b
      wait_out(j, b)

  return gather_kernel


def kernel(x, table):
  batch, hist = x.shape
  table128 = jnp.pad(table, ((0, 0), (0, PAD_DIM - EMBED_DIM)))
  return _make_gather(batch, hist)(x, table128)


# speedup vs baseline: 2.0010x; 1.0740x over previous
import functools

import jax
import jax.numpy as jnp
from jax import lax
from jax.experimental import pallas as pl
from jax.experimental.pallas import tpu as pltpu
from jax.experimental.pallas import tpu_sc as plsc

EMBED_DIM = 64
LANES = 16
NUM_CORES = 2
NUM_SUBCORES = 16
NW = NUM_CORES * NUM_SUBCORES
EPW = EMBED_DIM // NW

_mesh = plsc.VectorSubcoreMesh(core_axis_name="c", subcore_axis_name="s")


def _make_gather(batch: int, hist: int, vocab: int):
  nchunk = batch // LANES
  assert hist % 2 == 0

  @functools.partial(
      pl.kernel,
      mesh=_mesh,
      compiler_params=pltpu.CompilerParams(needs_layout_passes=False),
      out_type=jax.ShapeDtypeStruct((hist, EMBED_DIM, batch), jnp.float32),
      scratch_types=[
          pltpu.VMEM((vocab,), jnp.float32),
          pltpu.VMEM((2, batch), jnp.int32),
          pltpu.VMEM((2, batch), jnp.float32),
      ]
      + [pltpu.SemaphoreType.DMA] * 4,
  )
  def gather_kernel(xt_hbm, tt_hbm, out_hbm, rowbuf, idxbuf, outbuf, *sems):
    isem = sems[:2]
    osem = sems[2:]
    wid = lax.axis_index("s") * NUM_CORES + lax.axis_index("c")

    def fire_idx(h, p):
      pltpu.async_copy(xt_hbm.at[h], idxbuf.at[p], isem[p])

    def wait_idx(h, p):
      pltpu.make_async_copy(xt_hbm.at[h], idxbuf.at[p], isem[p]).wait()

    def fire_out(h, e, p):
      pltpu.async_copy(outbuf.at[p], out_hbm.at[h, e], osem[p])

    def wait_out(h, e, p):
      pltpu.make_async_copy(outbuf.at[p], out_hbm.at[h, e], osem[p]).wait()

    for ei in range(EPW):
      e = wid * EPW + ei
      pltpu.sync_copy(tt_hbm.at[e], rowbuf)
      fire_idx(0, 0)

      def body(g, carry, ei=ei, e=e):
        for p in range(2):
          h = g * 2 + p
          pl.when(h + 1 <= hist - 1)(lambda: fire_idx(h + 1, (p + 1) % 2))
          wait_idx(h, p)
          if ei == 0:
            pl.when(g >= 1)(lambda: wait_out(h - 2, e, p))
          else:
            pl.when(g >= 1)(lambda: wait_out(h - 2, e, p))
            pl.when(g == 0)(lambda: wait_out(hist - 2 + p, e - 1, p))
          for k in range(nchunk):
            sl = pl.ds(k * LANES, LANES)
            outbuf[p, sl] = plsc.load_gather(rowbuf, [idxbuf[p, sl]])
          fire_out(h, e, p)
        return carry

      lax.fori_loop(0, hist // 2, body, 0, unroll=False)

    e_last = wid * EPW + EPW - 1
    for p in range(2):
      wait_out(hist - 2 + p, e_last, p)

  return gather_kernel


def kernel(x, table):
  batch, hist = x.shape
  vocab, _ = table.shape
  out = _make_gather(batch, hist, vocab)(x.T, table.T)
  return out.transpose(2, 0, 1)


# speedup vs baseline: 3.3481x; 1.6732x over previous
import functools

import jax
import jax.numpy as jnp
from jax import lax
from jax.experimental import pallas as pl
from jax.experimental.pallas import tpu as pltpu
from jax.experimental.pallas import tpu_sc as plsc

EMBED_DIM = 64
LANES = 16
NUM_CORES = 2
NUM_SUBCORES = 16
NW = NUM_CORES * NUM_SUBCORES
EPW = EMBED_DIM // NW

_mesh = plsc.VectorSubcoreMesh(core_axis_name="c", subcore_axis_name="s")


def _make_gather(batch: int, hist: int, vocab: int):
  nchunk = batch // LANES
  assert hist % 2 == 0

  @functools.partial(
      pl.kernel,
      mesh=_mesh,
      compiler_params=pltpu.CompilerParams(needs_layout_passes=False),
      out_type=jax.ShapeDtypeStruct((hist, EMBED_DIM, batch), jnp.float32),
      scratch_types=[
          pltpu.VMEM((vocab,), jnp.float32),
          pltpu.VMEM((2, batch), jnp.int32),
          pltpu.VMEM((2, batch), jnp.float32),
      ]
      + [pltpu.SemaphoreType.DMA] * 4,
  )
  def gather_kernel(xt_hbm, tt_hbm, out_hbm, rowbuf, idxbuf, outbuf, *sems):
    isem = sems[:2]
    osem = sems[2:]
    wid = lax.axis_index("s") * NUM_CORES + lax.axis_index("c")

    def fire_idx(h, p):
      pltpu.async_copy(xt_hbm.at[h], idxbuf.at[p], isem[p])

    def wait_idx(h, p):
      pltpu.make_async_copy(xt_hbm.at[h], idxbuf.at[p], isem[p]).wait()

    def fire_out(h, e, p):
      pltpu.async_copy(outbuf.at[p], out_hbm.at[h, e], osem[p])

    def wait_out(h, e, p):
      pltpu.make_async_copy(outbuf.at[p], out_hbm.at[h, e], osem[p]).wait()

    for ei in range(EPW):
      e = wid * EPW + ei
      pltpu.sync_copy(tt_hbm.at[e], rowbuf)
      fire_idx(0, 0)

      def body(g, carry, ei=ei, e=e):
        for p in range(2):
          h = g * 2 + p
          pl.when(h + 1 <= hist - 1)(lambda: fire_idx(h + 1, (p + 1) % 2))
          wait_idx(h, p)
          if ei == 0:
            pl.when(g >= 1)(lambda: wait_out(h - 2, e, p))
          else:
            pl.when(g >= 1)(lambda: wait_out(h - 2, e, p))
            pl.when(g == 0)(lambda: wait_out(hist - 2 + p, e - 1, p))
          G = 8
          for kg in range(nchunk // G):
            sls = [pl.ds((kg * G + i) * LANES, LANES) for i in range(G)]
            idxs = [idxbuf[p, sl] for sl in sls]
            vals = [plsc.load_gather(rowbuf, [ix]) for ix in idxs]
            for sl, v in zip(sls, vals):
              outbuf[p, sl] = v
          fire_out(h, e, p)
        return carry

      lax.fori_loop(0, hist // 2, body, 0, unroll=False)

    e_last = wid * EPW + EPW - 1
    for p in range(2):
      wait_out(hist - 2 + p, e_last, p)

  return gather_kernel


def kernel(x, table):
  batch, hist = x.shape
  vocab, _ = table.shape
  out = _make_gather(batch, hist, vocab)(x.T, table.T)
  return out.transpose(2, 0, 1)
